# baseline (speedup 1.0000x reference)
import functools

import jax
import jax.numpy as jnp
import numpy as np
from jax import lax
from jax.experimental import pallas as pl
from jax.experimental.pallas import tpu as pltpu
from jax.experimental.pallas import tpu_sc as plsc

N = 10000
D = 256
DH = 128
E = 160000
NC = 2
NS = 16
EPS = E // NS
K = 80
NCHUNK = EPS // K
SUB = 8
SW = 16
NR = N * SUB
RA = NR // NS
NPC = 10240
RC = NPC // NS
ZR = 200


def _proj_body(x_ref, w_ref, b_ref, o_ref):
    o_ref[...] = (
        jnp.dot(x_ref[...], w_ref[...], preferred_element_type=jnp.float32)
        + b_ref[...]
    )


def _project(x, W, b):
    brm = 1000
    return pl.pallas_call(
        _proj_body,
        grid=(N // brm, 2),
        in_specs=[
            pl.BlockSpec((brm, D), lambda i, c: (i, 0)),
            pl.BlockSpec((D, DH), lambda i, c: (0, c)),
            pl.BlockSpec((1, DH), lambda i, c: (0, c)),
        ],
        out_specs=pl.BlockSpec((brm, DH), lambda i, c: (c * (N // brm) + i, 0)),
        out_shape=jax.ShapeDtypeStruct((2 * N, DH), jnp.float32),
    )(x, W, b.reshape(1, D))


def _gather_body(hid_u, hid_i, srcA_u2i, srcA_i2u,
                 msg_u2i_out, msg_i2u_out, idx_v, rows, sem):
    c = lax.axis_index("c")
    s = lax.axis_index("s")

    def do_etype(hid_hbm, srcA, msg_out):
        @pl.loop(0, NCHUNK)
        def chunk(g):
            off = pl.multiple_of(c * E + s * EPS + g * K, 16)
            pltpu.sync_copy(srcA.at[pl.ds(off, K)], idx_v)
            pltpu.async_copy(hid_hbm.at[idx_v], rows, sem).wait()
            pltpu.sync_copy(rows, msg_out.at[pl.ds(off, K)])

    do_etype(hid_u, srcA_u2i, msg_u2i_out)
    do_etype(hid_i, srcA_i2u, msg_i2u_out)


@functools.partial(jax.jit, static_argnums=())
def _gather_msgs(hid_u, hid_i, srcA_u2i, srcA_i2u):
    mesh = plsc.VectorSubcoreMesh(core_axis_name="c", subcore_axis_name="s")
    f = pl.kernel(
        _gather_body,
        out_type=[
            jax.ShapeDtypeStruct((2 * E, DH), jnp.float32),
            jax.ShapeDtypeStruct((2 * E, DH), jnp.float32),
        ],
        mesh=mesh,
        scratch_types=[
            pltpu.VMEM((K,), jnp.int32),
            pltpu.VMEM((K, DH), jnp.float32),
            pltpu.SemaphoreType.DMA,
        ],
    )
    return f(hid_u, hid_i, srcA_u2i, srcA_i2u)


def _scatter_body(msgT_u2i, msgT_i2u, dst_u2i, dst_i2u, zeros_z,
                  s_item_out, s_user_out, idx_dst, msgbuf, zbuf, acc):
    c = lax.axis_index("c")
    s = lax.axis_index("s")
    base = s * RC

    pltpu.sync_copy(zeros_z, zbuf)

    def do_pass(msgT, dst_hbm, p, s_out):
        for q in range(RC // K):
            pltpu.sync_copy(zbuf, acc.at[pl.ds(base + q * K, K)])
        plsc.subcore_barrier()

        def cloop(g, carry):
            eoff = pl.multiple_of(s * EPS + g * K, 16)
            pltpu.sync_copy(dst_hbm.at[pl.ds(eoff, K)], idx_dst)
            pltpu.sync_copy(msgT.at[p].at[pl.ds(c * E + eoff, K)], msgbuf)
            pltpu.sync_copy(msgbuf, acc.at[idx_dst], add=True)
            return carry

        lax.fori_loop(0, NCHUNK, cloop, 0)
        plsc.subcore_barrier()

        for q in range(RC // K):
            r = base + q * K
            pltpu.sync_copy(acc.at[pl.ds(r, K)], msgbuf)
            pltpu.sync_copy(msgbuf,
                            s_out.at[pl.ds((c * SUB + p) * NPC + r, K)])
        plsc.subcore_barrier()

    for p in range(SUB):
        do_pass(msgT_u2i, dst_u2i, p, s_item_out)
    for p in range(SUB):
        do_pass(msgT_i2u, dst_i2u, p, s_user_out)


@functools.partial(jax.jit, static_argnums=())
def _scatter_msgs(msgT_u2i, msgT_i2u, dst_u2i, dst_i2u, zeros_z):
    mesh = plsc.VectorSubcoreMesh(core_axis_name="c", subcore_axis_name="s")
    f = pl.kernel(
        _scatter_body,
        out_type=[
            jax.ShapeDtypeStruct((2 * SUB * NPC, SW), jnp.float32),
            jax.ShapeDtypeStruct((2 * SUB * NPC, SW), jnp.float32),
        ],
        mesh=mesh,
        scratch_types=[
            pltpu.VMEM((K,), jnp.int32),
            pltpu.VMEM((K, SW), jnp.float32),
            pltpu.VMEM((K, SW), jnp.float32),
            pltpu.VMEM_SHARED((NPC, SW), jnp.float32),
        ],
    )
    return f(msgT_u2i, msgT_i2u, dst_u2i, dst_i2u, zeros_z)


def _count_body(dst_u2i, dst_i2u, count_src, zeros_cnt,
                cnt_item_out, cnt_user_out, idx_dst, csrc, stage_cnt, cntacc):
    c = lax.axis_index("c")
    s = lax.axis_index("s")
    base_cnt = s * RC

    pltpu.sync_copy(count_src, csrc)
    pltpu.sync_copy(zeros_cnt, stage_cnt)

    def do_etype(dst_hbm, cnt_out):
        for q in range(RC // K):
            pltpu.sync_copy(stage_cnt, cntacc.at[pl.ds(base_cnt + q * K, K)])
        plsc.subcore_barrier()

        def cloop(g, carry):
            eoff = pl.multiple_of(s * EPS + g * K, 16)
            pltpu.sync_copy(dst_hbm.at[pl.ds(eoff, K)], idx_dst)
            pltpu.sync_copy(csrc, cntacc.at[idx_dst], add=True)
            return carry

        lax.fori_loop(0, NCHUNK, cloop, 0)
        plsc.subcore_barrier()
        for q in range(RC // K):
            r = base_cnt + q * K
            pltpu.sync_copy(cntacc.at[pl.ds(r, K)], stage_cnt)
            pltpu.sync_copy(stage_cnt, cnt_out.at[pl.ds(c * NPC + r, K)])
        plsc.subcore_barrier()
        pltpu.sync_copy(zeros_cnt, stage_cnt)

    do_etype(dst_u2i, cnt_item_out)
    do_etype(dst_i2u, cnt_user_out)


@functools.partial(jax.jit, static_argnums=())
def _count_msgs(dst_u2i, dst_i2u, count_src, zeros_cnt):
    mesh = plsc.VectorSubcoreMesh(core_axis_name="c", subcore_axis_name="s")
    f = pl.kernel(
        _count_body,
        out_type=[
            jax.ShapeDtypeStruct((2 * NPC, SW), jnp.float32),
            jax.ShapeDtypeStruct((2 * NPC, SW), jnp.float32),
        ],
        mesh=mesh,
        scratch_types=[
            pltpu.VMEM((K,), jnp.int32),
            pltpu.VMEM((K, SW), jnp.float32),
            pltpu.VMEM((K, SW), jnp.float32),
            pltpu.VMEM_SHARED((NPC, SW), jnp.float32),
        ],
    )
    return f(dst_u2i, dst_i2u, count_src, zeros_cnt)


def _finish_body(s_ref, cnt_ref, hid_ref, o_ref):
    cnt = cnt_ref[...][:, 0:1]
    o_ref[...] = hid_ref[...] + s_ref[...] / jnp.maximum(cnt, 1.0)


def _finish(s_s, cnt_n, hid_s):
    brf = 1000
    return pl.pallas_call(
        _finish_body,
        grid=(N // brf, 2),
        in_specs=[
            pl.BlockSpec((brf, DH), lambda i, c: (i, c)),
            pl.BlockSpec((brf, SW), lambda i, c: (i, 0)),
            pl.BlockSpec((brf, DH), lambda i, c: (c * (N // brf) + i, 0)),
        ],
        out_specs=pl.BlockSpec((brf, DH), lambda i, c: (i, c)),
        out_shape=jax.ShapeDtypeStruct((N, D), jnp.float32),
    )(s_s, cnt_n, hid_s)


def kernel(x_user, x_item, W_user, b_user, W_item, b_item,
           edge_index_u2i, edge_index_i2u):
    hid_user_s = _project(x_user.astype(jnp.float32), W_user, b_user)
    hid_item_s = _project(x_item.astype(jnp.float32), W_item, b_item)

    e_u2i = edge_index_u2i.astype(jnp.int32)
    e_i2u = edge_index_i2u.astype(jnp.int32)

    srcA_u2i = jnp.concatenate([e_u2i[0], e_u2i[0] + N])
    srcA_i2u = jnp.concatenate([e_i2u[0], e_i2u[0] + N])

    count_np = np.zeros((K, SW), np.float32)
    count_np[:, 0] = 1.0
    count_src = jnp.asarray(count_np)
    zeros_z = jnp.asarray(np.zeros((K, SW), np.float32))
    zeros_cnt = jnp.asarray(np.zeros((K, SW), np.float32))

    msg_u2i, msg_i2u = _gather_msgs(hid_user_s, hid_item_s,
                                    srcA_u2i, srcA_i2u)

    msgT_u2i = msg_u2i.reshape(2 * E, SUB, SW).transpose(1, 0, 2)
    msgT_i2u = msg_i2u.reshape(2 * E, SUB, SW).transpose(1, 0, 2)

    s_item_r, s_user_r = _scatter_msgs(msgT_u2i, msgT_i2u,
                                       e_u2i[1], e_i2u[1], zeros_z)
    cnt_item_r, cnt_user_r = _count_msgs(e_u2i[1], e_i2u[1],
                                         count_src, zeros_cnt)

    s_item_m = (s_item_r.reshape(2 * SUB, NPC, SW)[:, :N]
                .transpose(1, 0, 2).reshape(N, D))
    s_user_m = (s_user_r.reshape(2 * SUB, NPC, SW)[:, :N]
                .transpose(1, 0, 2).reshape(N, D))

    out_user = _finish(s_user_m, cnt_user_r[:N], hid_user_s)
    out_item = _finish(s_item_m, cnt_item_r[:N], hid_item_s)
    return (out_user, out_item)
